# baseline (speedup 1.0000x reference)
import jax
import jax.numpy as jnp
from jax import lax
from jax.experimental import pallas as pl
from jax.experimental.pallas import tpu as pltpu
from jax.experimental.pallas import tpu_sc as plsc


def _sc_body(x_hbm, w_hbm, b_hbm, tok_hbm, out_hbm, x_v, w_v, b_v, tok_v, out_v):
    cid = lax.axis_index("c")
    sid = lax.axis_index("s")

    @pl.when(jnp.logical_and(cid == 0, sid == 0))
    def _():
        pltpu.sync_copy(x_hbm, x_v)
        pltpu.sync_copy(w_hbm, w_v)
        pltpu.sync_copy(b_hbm, b_v)
        pltpu.sync_copy(tok_hbm, tok_v.at[pl.ds(0, 8)])
        xvec = x_v[:]
        tokvec = tok_v[:]
        acc0 = b_v[:]
        acc1 = b_v[:]
        for k in range(8):
            col = w_v[k, :]
            acc0 = acc0 + tokvec[k] * col
            acc1 = acc1 + xvec[8 + k] * col
        out_v[0, :] = acc0
        out_v[1, :] = acc1
        pltpu.sync_copy(out_v, out_hbm)


def kernel(x, W, b, token):
    mesh = plsc.VectorSubcoreMesh(core_axis_name="c", subcore_axis_name="s")
    x_flat = x.reshape(-1)
    w_t = W.T
    f = pl.kernel(
        _sc_body,
        out_type=jax.ShapeDtypeStruct((2, 16), jnp.float32),
        mesh=mesh,
        scratch_types=[
            pltpu.VMEM((16,), jnp.float32),
            pltpu.VMEM((8, 16), jnp.float32),
            pltpu.VMEM((16,), jnp.float32),
            pltpu.VMEM((16,), jnp.float32),
            pltpu.VMEM((2, 16), jnp.float32),
        ],
    )
    return f(x_flat, w_t, b, token)


# speedup vs baseline: 1.1303x; 1.1303x over previous
import jax
import jax.numpy as jnp
from jax import lax
from jax.experimental import pallas as pl
from jax.experimental.pallas import tpu as pltpu
from jax.experimental.pallas import tpu_sc as plsc


def _sc_body(x_hbm, w_hbm, b_hbm, tok_hbm, out_hbm, x_v, w_v, b_v, tok_v, out_v,
             sem):
    sid = lax.axis_index("s")

    @pl.when(sid == 0)
    def _():
        c1 = pltpu.async_copy(x_hbm, x_v, sem)
        c2 = pltpu.async_copy(w_hbm, w_v, sem)
        c3 = pltpu.async_copy(b_hbm, b_v, sem)
        c4 = pltpu.async_copy(tok_hbm, tok_v.at[pl.ds(0, 8)], sem)
        c1.wait()
        c2.wait()
        c3.wait()
        c4.wait()
        xvec = x_v[:]
        tokvec = tok_v[:]
        acc0 = b_v[:]
        acc1 = b_v[:]
        for k in range(8):
            col = w_v[k, :]
            acc0 = acc0 + tokvec[k] * col
            acc1 = acc1 + xvec[8 + k] * col
        out_v[0, :] = acc0
        out_v[1, :] = acc1
        pltpu.sync_copy(out_v, out_hbm)


def kernel(x, W, b, token):
    mesh = plsc.VectorSubcoreMesh(
        core_axis_name="c", subcore_axis_name="s", num_cores=1
    )
    x_flat = x.reshape(-1)
    w_t = W.T
    f = pl.kernel(
        _sc_body,
        out_type=jax.ShapeDtypeStruct((2, 16), jnp.float32),
        mesh=mesh,
        scratch_types=[
            pltpu.VMEM((16,), jnp.float32),
            pltpu.VMEM((8, 16), jnp.float32),
            pltpu.VMEM((16,), jnp.float32),
            pltpu.VMEM((16,), jnp.float32),
            pltpu.VMEM((2, 16), jnp.float32),
            pltpu.SemaphoreType.DMA,
        ],
    )
    return f(x_flat, w_t, b, token)
